# baseline (speedup 1.0000x reference)
import functools

import jax
import jax.numpy as jnp
from jax import lax
from jax.experimental import pallas as pl
from jax.experimental.pallas import tpu as pltpu
from jax.experimental.pallas import tpu_sc as plsc

NC = 2
NS = 16
DUMP = 2048
PIX = 512


def _prep_kernel(im0_ref, flow_ref, idx_ref, wts_ref, imr_ref, *, hblk, W, H):
    hb = pl.program_id(1)
    half = (H // 2) * W
    nr = hblk * W // 128
    wt = W // 128
    f3 = flow_ref[0].reshape(nr, 2, 128)
    fx = f3[:, 0]
    fy = f3[:, 1]
    r_i = lax.broadcasted_iota(jnp.int32, (nr, 128), 0)
    l_i = lax.broadcasted_iota(jnp.int32, (nr, 128), 1)
    gxi = lax.rem(r_i, wt) * 128 + l_i
    gyi = r_i // wt + hb * hblk
    x = gxi.astype(jnp.float32) + fx
    y = gyi.astype(jnp.float32) + fy
    x0 = jnp.floor(x)
    y0 = jnp.floor(y)
    frx = x - x0
    fry = y - y0
    p = gyi * W + gxi
    dump = half + jnp.bitwise_and(p, DUMP - 1)

    for c in range(8):
        imr_ref[0, c] = im0_ref[0, c].reshape(nr, 128)

    k = 0
    for h in (0, 1):
        iy = y0 + h
        wy = fry if h else (1.0 - fry)
        iyi = jnp.clip(iy.astype(jnp.int32), 0, H - 1)
        yok = (iy >= 0) & (iy < H)
        for s in (0, 1):
            ix = x0 + s
            wx = frx if s else (1.0 - frx)
            ixi = jnp.clip(ix.astype(jnp.int32), 0, W - 1)
            ok = yok & (ix >= 0) & (ix < W)
            wts_ref[0, k] = jnp.where(ok, wx * wy, 0.0)
            gidx = iyi * W + ixi
            idx_ref[0, 0, k] = jnp.where(gidx < half, gidx, dump)
            idx_ref[1, 0, k] = jnp.where(gidx >= half, gidx - half, dump)
            k += 1


def _post_kernel(acc_ref, out_ref, *, hblk, W):
    m = hblk * W * 8 // 128
    a = acc_ref[0].reshape(m, 16, 8)
    t = jnp.transpose(a, (2, 0, 1)).reshape(8, m * 16)
    for c in range(8):
        out_ref[0, c] = t[c].reshape(hblk, W)


def _sc_scatter(im_hbm, wts_hbm, idx_hbm, zeros_hbm, out_hbm, acc, imbuf,
                wbuf, ibuf, stage, sem_in, *, B, HW):
    core = lax.axis_index("c")
    sid = lax.axis_index("s")
    half = HW // 2
    slc = half // NS
    t_pix = HW // NS
    nchunk = t_pix // PIX
    my0 = pl.multiple_of(sid * slc, slc)
    iota = lax.iota(jnp.int32, 16)
    cols = [jnp.full((16,), c, jnp.int32) for c in range(8)]

    @pl.loop(0, B)
    def _round(b):
        pltpu.sync_copy(zeros_hbm, acc.at[pl.ds(my0, slc)])
        plsc.subcore_barrier()

        @pl.loop(0, nchunk)
        def _chunk(q):
            p0 = pl.multiple_of(sid * t_pix + q * PIX, PIX)
            r0 = pl.multiple_of(p0 // 128, PIX // 128)
            cp_m = pltpu.async_copy(
                im_hbm.at[b, :, pl.ds(r0, PIX // 128)], imbuf, sem_in)
            cp_w = pltpu.async_copy(
                wts_hbm.at[b, :, pl.ds(r0, PIX // 128)], wbuf, sem_in)
            cp_i = pltpu.async_copy(
                idx_hbm.at[core, b, :, pl.ds(r0, PIX // 128)], ibuf, sem_in)
            cp_m.wait()
            cp_w.wait()
            cp_i.wait()
            for g in range(PIX // 16):
                row, col = g // 8, (g % 8) * 16
                vcs = [imbuf[c, row, pl.ds(col, 16)] for c in range(8)]
                for k in range(4):
                    rows = iota + (k * PIX + g * 16)
                    wv = wbuf[k, row, pl.ds(col, 16)]
                    for c in range(8):
                        plsc.store_scatter(stage, [rows, cols[c]],
                                           vcs[c] * wv)
            for k in range(4):
                for j in range(PIX // 128):
                    pltpu.sync_copy(
                        stage.at[pl.ds(k * PIX + j * 128, 128)],
                        acc.at[ibuf.at[k, j]],
                        add=True,
                    )

        plsc.subcore_barrier()
        out0 = pl.multiple_of(core * half + sid * slc, slc)
        pltpu.sync_copy(
            acc.at[pl.ds(my0, slc)],
            out_hbm.at[b, pl.ds(out0, slc)],
        )
        plsc.subcore_barrier()


def kernel(im0, flow):
    B, C, H, W = im0.shape
    HW = H * W
    half = HW // 2
    hblk = 64
    n = hblk * W

    grid = (B, H // hblk)
    flowv = jnp.transpose(flow.reshape(B, H, W // 128, 128, 2),
                          (0, 1, 2, 4, 3)).reshape(B, H * (W // 128) * 2, 128)
    idx, wts, imr = pl.pallas_call(
        functools.partial(_prep_kernel, hblk=hblk, W=W, H=H),
        grid=grid,
        in_specs=[
            pl.BlockSpec((1, 8, hblk, W), lambda b, hb: (b, 0, hb, 0)),
            pl.BlockSpec((1, hblk * (W // 128) * 2, 128),
                         lambda b, hb: (b, hb, 0)),
        ],
        out_specs=[
            pl.BlockSpec((2, 1, 4, n // 128, 128),
                         lambda b, hb: (0, b, 0, hb, 0)),
            pl.BlockSpec((1, 4, n // 128, 128), lambda b, hb: (b, 0, hb, 0)),
            pl.BlockSpec((1, 8, n // 128, 128), lambda b, hb: (b, 0, hb, 0)),
        ],
        out_shape=[
            jax.ShapeDtypeStruct((2, B, 4, HW // 128, 128), jnp.int32),
            jax.ShapeDtypeStruct((B, 4, HW // 128, 128), jnp.float32),
            jax.ShapeDtypeStruct((B, 8, HW // 128, 128), jnp.float32),
        ],
    )(im0, flowv)

    zeros = jnp.zeros((half // NS, 8), jnp.float32)

    mesh = plsc.VectorSubcoreMesh(core_axis_name="c", subcore_axis_name="s")
    acc = pl.kernel(
        functools.partial(_sc_scatter, B=B, HW=HW),
        out_type=jax.ShapeDtypeStruct((B, HW, 8), jnp.float32),
        mesh=mesh,
        compiler_params=pltpu.CompilerParams(
            use_tc_tiling_on_sc=False, needs_layout_passes=False
        ),
        scratch_types=[
            pltpu.VMEM_SHARED((half + DUMP, 8), jnp.float32),
            pltpu.VMEM((8, PIX // 128, 128), jnp.float32),
            pltpu.VMEM((4, PIX // 128, 128), jnp.float32),
            pltpu.VMEM((4, PIX // 128, 128), jnp.int32),
            pltpu.VMEM((4 * PIX, 8), jnp.float32),
            pltpu.SemaphoreType.DMA,
        ],
    )(imr, wts, idx, zeros)

    out = jnp.transpose(acc.reshape(B, HW, 8), (0, 2, 1)).reshape(B, C, H, W)
    return out


# speedup vs baseline: 1.1562x; 1.1562x over previous
import functools

import jax
import jax.numpy as jnp
from jax import lax
from jax.experimental import pallas as pl
from jax.experimental.pallas import tpu as pltpu
from jax.experimental.pallas import tpu_sc as plsc

NC = 2
NS = 16
DUMP = 2048
PIX = 512


def _prep_kernel(im0_ref, flow_ref, idx_ref, wts_ref, imr_ref, *, hblk, W, H):
    hb = pl.program_id(1)
    half = (H // 2) * W
    nr = hblk * W // 128
    wt = W // 128
    f3 = flow_ref[0].reshape(nr, 2, 128)
    fx = f3[:, 0]
    fy = f3[:, 1]
    r_i = lax.broadcasted_iota(jnp.int32, (nr, 128), 0)
    l_i = lax.broadcasted_iota(jnp.int32, (nr, 128), 1)
    gxi = lax.rem(r_i, wt) * 128 + l_i
    gyi = r_i // wt + hb * hblk
    x = gxi.astype(jnp.float32) + fx
    y = gyi.astype(jnp.float32) + fy
    x0 = jnp.floor(x)
    y0 = jnp.floor(y)
    frx = x - x0
    fry = y - y0
    p = gyi * W + gxi
    dump = half + jnp.bitwise_and(p, DUMP - 1)

    for c in range(8):
        imr_ref[0, c] = im0_ref[0, c].reshape(nr, 128)

    k = 0
    for h in (0, 1):
        iy = y0 + h
        wy = fry if h else (1.0 - fry)
        iyi = jnp.clip(iy.astype(jnp.int32), 0, H - 1)
        yok = (iy >= 0) & (iy < H)
        for s in (0, 1):
            ix = x0 + s
            wx = frx if s else (1.0 - frx)
            ixi = jnp.clip(ix.astype(jnp.int32), 0, W - 1)
            ok = yok & (ix >= 0) & (ix < W)
            wts_ref[0, k] = jnp.where(ok, wx * wy, 0.0)
            gidx = iyi * W + ixi
            idx_ref[0, 0, k] = jnp.where(gidx < half, gidx, dump)
            idx_ref[1, 0, k] = jnp.where(gidx >= half, gidx - half, dump)
            k += 1


def _post_kernel(acc_ref, out_ref, *, hblk, W):
    m = hblk * W * 8 // 128
    a = acc_ref[0].reshape(m, 16, 8)
    t = jnp.transpose(a, (2, 0, 1)).reshape(8, m * 16)
    for c in range(8):
        out_ref[0, c] = t[c].reshape(hblk, W)


def _sc_scatter(im_hbm, wts_hbm, idx_hbm, zeros_hbm, out_hbm, acc, imbuf,
                wbuf, ibuf, stage, sem_in, sem_st, *, B, HW):
    core = lax.axis_index("c")
    sid = lax.axis_index("s")
    half = HW // 2
    slc = half // NS
    t_pix = HW // NS
    nchunk = t_pix // PIX
    my0 = pl.multiple_of(sid * slc, slc)
    iota = lax.iota(jnp.int32, 16)
    cols = [jnp.full((16,), c, jnp.int32) for c in range(8)]

    @pl.loop(0, B)
    def _round(b):
        pltpu.sync_copy(zeros_hbm, acc.at[pl.ds(my0, slc)])
        plsc.subcore_barrier()

        @pl.loop(0, nchunk)
        def _chunk(q):
            p0 = pl.multiple_of(sid * t_pix + q * PIX, PIX)
            r0 = pl.multiple_of(p0 // 128, PIX // 128)
            cp_m = pltpu.async_copy(
                im_hbm.at[b, :, pl.ds(r0, PIX // 128)], imbuf, sem_in)
            cp_w = pltpu.async_copy(
                wts_hbm.at[b, :, pl.ds(r0, PIX // 128)], wbuf, sem_in)
            cp_i = pltpu.async_copy(
                idx_hbm.at[core, b, :, pl.ds(r0, PIX // 128)], ibuf, sem_in)
            cp_m.wait()
            cp_w.wait()
            cp_i.wait()
            for g in range(PIX // 16):
                row, col = g // 8, (g % 8) * 16
                vcs = [imbuf[c, row, pl.ds(col, 16)] for c in range(8)]
                for k in range(4):
                    rows = iota + (k * PIX + g * 16)
                    wv = wbuf[k, row, pl.ds(col, 16)]
                    for c in range(8):
                        plsc.store_scatter(stage, [rows, cols[c]],
                                           vcs[c] * wv)
            cps = []
            for k in range(4):
                for j in range(PIX // 128):
                    cps.append(pltpu.async_copy(
                        stage.at[pl.ds(k * PIX + j * 128, 128)],
                        acc.at[ibuf.at[k, j]],
                        sem_st,
                        add=True,
                    ))
            for cp in cps:
                cp.wait()

        plsc.subcore_barrier()
        out0 = pl.multiple_of(core * half + sid * slc, slc)
        pltpu.sync_copy(
            acc.at[pl.ds(my0, slc)],
            out_hbm.at[b, pl.ds(out0, slc)],
        )
        plsc.subcore_barrier()


def kernel(im0, flow):
    B, C, H, W = im0.shape
    HW = H * W
    half = HW // 2
    hblk = 64
    n = hblk * W

    grid = (B, H // hblk)
    flowv = jnp.transpose(flow.reshape(B, H, W // 128, 128, 2),
                          (0, 1, 2, 4, 3)).reshape(B, H * (W // 128) * 2, 128)
    idx, wts, imr = pl.pallas_call(
        functools.partial(_prep_kernel, hblk=hblk, W=W, H=H),
        grid=grid,
        in_specs=[
            pl.BlockSpec((1, 8, hblk, W), lambda b, hb: (b, 0, hb, 0)),
            pl.BlockSpec((1, hblk * (W // 128) * 2, 128),
                         lambda b, hb: (b, hb, 0)),
        ],
        out_specs=[
            pl.BlockSpec((2, 1, 4, n // 128, 128),
                         lambda b, hb: (0, b, 0, hb, 0)),
            pl.BlockSpec((1, 4, n // 128, 128), lambda b, hb: (b, 0, hb, 0)),
            pl.BlockSpec((1, 8, n // 128, 128), lambda b, hb: (b, 0, hb, 0)),
        ],
        out_shape=[
            jax.ShapeDtypeStruct((2, B, 4, HW // 128, 128), jnp.int32),
            jax.ShapeDtypeStruct((B, 4, HW // 128, 128), jnp.float32),
            jax.ShapeDtypeStruct((B, 8, HW // 128, 128), jnp.float32),
        ],
    )(im0, flowv)

    zeros = jnp.zeros((half // NS, 8), jnp.float32)

    mesh = plsc.VectorSubcoreMesh(core_axis_name="c", subcore_axis_name="s")
    acc = pl.kernel(
        functools.partial(_sc_scatter, B=B, HW=HW),
        out_type=jax.ShapeDtypeStruct((B, HW, 8), jnp.float32),
        mesh=mesh,
        compiler_params=pltpu.CompilerParams(
            use_tc_tiling_on_sc=False, needs_layout_passes=False
        ),
        scratch_types=[
            pltpu.VMEM_SHARED((half + DUMP, 8), jnp.float32),
            pltpu.VMEM((8, PIX // 128, 128), jnp.float32),
            pltpu.VMEM((4, PIX // 128, 128), jnp.float32),
            pltpu.VMEM((4, PIX // 128, 128), jnp.int32),
            pltpu.VMEM((4 * PIX, 8), jnp.float32),
            pltpu.SemaphoreType.DMA,
            pltpu.SemaphoreType.DMA,
        ],
    )(imr, wts, idx, zeros)

    accv = acc.reshape(B, HW * 8 // 128, 128)
    out = pl.pallas_call(
        functools.partial(_post_kernel, hblk=hblk, W=W),
        grid=grid,
        in_specs=[
            pl.BlockSpec((1, n * 8 // 128, 128), lambda b, hb: (b, hb, 0)),
        ],
        out_specs=pl.BlockSpec((1, 8, hblk, W), lambda b, hb: (b, 0, hb, 0)),
        out_shape=jax.ShapeDtypeStruct((B, C, H, W), jnp.float32),
    )(accv)
    return out


# speedup vs baseline: 1.5761x; 1.3631x over previous
import functools

import jax
import jax.numpy as jnp
from jax import lax
from jax.experimental import pallas as pl
from jax.experimental.pallas import tpu as pltpu
from jax.experimental.pallas import tpu_sc as plsc

NC = 2
NS = 16
DUMP = 2048
PIX = 512


def _prep_kernel(im0_ref, flow_ref, idx_ref, wts_ref, imr_ref, *, hblk, W, H):
    hb = pl.program_id(1)
    half = (H // 2) * W
    nr = hblk * W // 128
    wt = W // 128
    f3 = flow_ref[0].reshape(nr, 2, 128)
    fx = f3[:, 0]
    fy = f3[:, 1]
    r_i = lax.broadcasted_iota(jnp.int32, (nr, 128), 0)
    l_i = lax.broadcasted_iota(jnp.int32, (nr, 128), 1)
    gxi = lax.rem(r_i, wt) * 128 + l_i
    gyi = r_i // wt + hb * hblk
    x = gxi.astype(jnp.float32) + fx
    y = gyi.astype(jnp.float32) + fy
    x0 = jnp.floor(x)
    y0 = jnp.floor(y)
    frx = x - x0
    fry = y - y0
    p = gyi * W + gxi
    dump = half + jnp.bitwise_and(p, DUMP - 1)

    for c in range(8):
        imr_ref[0, c] = im0_ref[0, c].reshape(nr, 128)

    k = 0
    for h in (0, 1):
        iy = y0 + h
        wy = fry if h else (1.0 - fry)
        iyi = jnp.clip(iy.astype(jnp.int32), 0, H - 1)
        yok = (iy >= 0) & (iy < H)
        for s in (0, 1):
            ix = x0 + s
            wx = frx if s else (1.0 - frx)
            ixi = jnp.clip(ix.astype(jnp.int32), 0, W - 1)
            ok = yok & (ix >= 0) & (ix < W)
            wts_ref[0, k] = jnp.where(ok, wx * wy, 0.0)
            gidx = iyi * W + ixi
            idx_ref[0, 0, k] = jnp.where(gidx < half, gidx, dump)
            idx_ref[1, 0, k] = jnp.where(gidx >= half, gidx - half, dump)
            k += 1


def _post_kernel(acc_ref, out_ref, *, hblk, W):
    m = hblk * W * 8 // 128
    a = acc_ref[0].reshape(m, 16, 8)
    t = jnp.transpose(a, (2, 0, 1)).reshape(8, m * 16)
    for c in range(8):
        out_ref[0, c] = t[c].reshape(hblk, W)


def _sc_scatter(im_hbm, wts_hbm, idx_hbm, zeros_hbm, out_hbm, acc, imbuf,
                wbuf, ibuf, stage, sem_i0, sem_i1, sem_s0, sem_s1, *, B, HW):
    core = lax.axis_index("c")
    sid = lax.axis_index("s")
    half = HW // 2
    slc = half // NS
    t_pix = HW // NS
    nchunk = t_pix // PIX
    my0 = pl.multiple_of(sid * slc, slc)
    iota = lax.iota(jnp.int32, 16)
    cols = [jnp.full((16,), c, jnp.int32) for c in range(8)]
    sem_i = (sem_i0, sem_i1)
    sem_s = (sem_s0, sem_s1)

    def issue_inputs(b, q, p):
        r0 = pl.multiple_of((sid * t_pix + q * PIX) // 128, PIX // 128)
        pltpu.async_copy(
            im_hbm.at[b, :, pl.ds(r0, PIX // 128)], imbuf.at[p], sem_i[p])
        pltpu.async_copy(
            wts_hbm.at[b, :, pl.ds(r0, PIX // 128)], wbuf.at[p], sem_i[p])
        pltpu.async_copy(
            idx_hbm.at[core, b, :, pl.ds(r0, PIX // 128)], ibuf.at[p],
            sem_i[p])

    def wait_inputs(b, q, p):
        r0 = pl.multiple_of((sid * t_pix + q * PIX) // 128, PIX // 128)
        pltpu.make_async_copy(
            im_hbm.at[b, :, pl.ds(r0, PIX // 128)], imbuf.at[p],
            sem_i[p]).wait()
        pltpu.make_async_copy(
            wts_hbm.at[b, :, pl.ds(r0, PIX // 128)], wbuf.at[p],
            sem_i[p]).wait()
        pltpu.make_async_copy(
            idx_hbm.at[core, b, :, pl.ds(r0, PIX // 128)], ibuf.at[p],
            sem_i[p]).wait()

    def drain_streams(sem):
        for _ in range(16):
            pltpu.make_async_copy(
                zeros_hbm.at[pl.ds(0, 128)], acc.at[pl.ds(0, 128)],
                sem).wait()

    @pl.loop(0, B)
    def _round(b):
        pltpu.sync_copy(zeros_hbm, acc.at[pl.ds(my0, slc)])
        plsc.subcore_barrier()
        issue_inputs(b, 0, 0)

        @pl.loop(0, nchunk // 2)
        def _chunk2(qq):
            for par in (0, 1):
                q = qq * 2 + par
                oth = 1 - par
                wait_inputs(b, q, par)
                st = stage.at[par]

                @pl.loop(0, PIX // 16)
                def _build(g):
                    row = g // 8
                    col = pl.multiple_of(lax.rem(g, 8) * 16, 16)
                    vcs = [imbuf[par, c, row, pl.ds(col, 16)]
                           for c in range(8)]
                    for k in range(4):
                        rows = iota + (k * PIX + g * 16)
                        wv = wbuf[par, k, row, pl.ds(col, 16)]
                        for c in range(8):
                            plsc.store_scatter(st, [rows, cols[c]],
                                               vcs[c] * wv)
                if par == 0:
                    @pl.when(qq >= 1)
                    def _():
                        drain_streams(sem_s[oth])
                else:
                    drain_streams(sem_s[oth])
                if par == 0:
                    issue_inputs(b, q + 1, oth)
                else:
                    @pl.when(qq < nchunk // 2 - 1)
                    def _():
                        issue_inputs(b, q + 1, oth)
                for k in range(4):
                    for j in range(PIX // 128):
                        pltpu.async_copy(
                            st.at[pl.ds(k * PIX + j * 128, 128)],
                            acc.at[ibuf.at[par, k, j]],
                            sem_s[par],
                            add=True,
                        )

        drain_streams(sem_s[1])
        plsc.subcore_barrier()
        out0 = pl.multiple_of(core * half + sid * slc, slc)
        pltpu.sync_copy(
            acc.at[pl.ds(my0, slc)],
            out_hbm.at[b, pl.ds(out0, slc)],
        )
        plsc.subcore_barrier()


def kernel(im0, flow):
    B, C, H, W = im0.shape
    HW = H * W
    half = HW // 2
    hblk = 64
    n = hblk * W

    grid = (B, H // hblk)
    flowv = jnp.transpose(flow.reshape(B, H, W // 128, 128, 2),
                          (0, 1, 2, 4, 3)).reshape(B, H * (W // 128) * 2, 128)
    idx, wts, imr = pl.pallas_call(
        functools.partial(_prep_kernel, hblk=hblk, W=W, H=H),
        grid=grid,
        in_specs=[
            pl.BlockSpec((1, 8, hblk, W), lambda b, hb: (b, 0, hb, 0)),
            pl.BlockSpec((1, hblk * (W // 128) * 2, 128),
                         lambda b, hb: (b, hb, 0)),
        ],
        out_specs=[
            pl.BlockSpec((2, 1, 4, n // 128, 128),
                         lambda b, hb: (0, b, 0, hb, 0)),
            pl.BlockSpec((1, 4, n // 128, 128), lambda b, hb: (b, 0, hb, 0)),
            pl.BlockSpec((1, 8, n // 128, 128), lambda b, hb: (b, 0, hb, 0)),
        ],
        out_shape=[
            jax.ShapeDtypeStruct((2, B, 4, HW // 128, 128), jnp.int32),
            jax.ShapeDtypeStruct((B, 4, HW // 128, 128), jnp.float32),
            jax.ShapeDtypeStruct((B, 8, HW // 128, 128), jnp.float32),
        ],
    )(im0, flowv)

    zeros = jnp.zeros((half // NS, 8), jnp.float32)

    mesh = plsc.VectorSubcoreMesh(core_axis_name="c", subcore_axis_name="s")
    acc = pl.kernel(
        functools.partial(_sc_scatter, B=B, HW=HW),
        out_type=jax.ShapeDtypeStruct((B, HW, 8), jnp.float32),
        mesh=mesh,
        compiler_params=pltpu.CompilerParams(
            use_tc_tiling_on_sc=False, needs_layout_passes=False
        ),
        scratch_types=[
            pltpu.VMEM_SHARED((half + DUMP, 8), jnp.float32),
            pltpu.VMEM((2, 8, PIX // 128, 128), jnp.float32),
            pltpu.VMEM((2, 4, PIX // 128, 128), jnp.float32),
            pltpu.VMEM((2, 4, PIX // 128, 128), jnp.int32),
            pltpu.VMEM((2, 4 * PIX, 8), jnp.float32),
            pltpu.SemaphoreType.DMA,
            pltpu.SemaphoreType.DMA,
            pltpu.SemaphoreType.DMA,
            pltpu.SemaphoreType.DMA,
        ],
    )(imr, wts, idx, zeros)

    accv = acc.reshape(B, HW * 8 // 128, 128)
    out = pl.pallas_call(
        functools.partial(_post_kernel, hblk=hblk, W=W),
        grid=grid,
        in_specs=[
            pl.BlockSpec((1, n * 8 // 128, 128), lambda b, hb: (b, hb, 0)),
        ],
        out_specs=pl.BlockSpec((1, 8, hblk, W), lambda b, hb: (b, 0, hb, 0)),
        out_shape=jax.ShapeDtypeStruct((B, C, H, W), jnp.float32),
    )(accv)
    return out
